# baseline (speedup 1.0000x reference)
import functools

import jax
import jax.numpy as jnp
from jax import lax
from jax.experimental import pallas as pl
from jax.experimental.pallas import tpu as pltpu
from jax.experimental.pallas import tpu_sc as plsc


def _make_agg(N, D, E):
    info = plsc.get_sparse_core_info()
    NC, NS = info.num_cores, info.num_subcores
    NW = NC * NS
    CH = 128
    CPT = 80
    n_total = NW * CPT
    assert n_total * CH >= E
    ZR = 80
    n_row_chunks = N // ZR
    assert n_row_chunks * ZR == N
    zchunks_per_tile = -(-n_row_chunks // NS)

    mesh = plsc.VectorSubcoreMesh(core_axis_name="c", subcore_axis_name="s")

    @functools.partial(
        pl.kernel,
        out_type=jax.ShapeDtypeStruct((NC, N, D), jnp.float32),
        mesh=mesh,
        scratch_types=[
            pltpu.VMEM((CH,), jnp.int32),
            pltpu.VMEM((CH,), jnp.int32),
            pltpu.VMEM((CH,), jnp.int32),
            pltpu.VMEM((CH,), jnp.int32),
            pltpu.VMEM((CH,), jnp.int32),
            pltpu.VMEM((CH,), jnp.int32),
            pltpu.VMEM((CH,), jnp.int32),
            pltpu.VMEM((CH,), jnp.int32),
            pltpu.VMEM((CH, D), jnp.float32),
            pltpu.VMEM((CH, D), jnp.float32),
            pltpu.VMEM_SHARED((N + 8, D), jnp.float32),
            pltpu.SemaphoreType.DMA,
            pltpu.SemaphoreType.DMA,
            pltpu.SemaphoreType.DMA,
            pltpu.SemaphoreType.DMA,
            pltpu.SemaphoreType.DMA,
            pltpu.SemaphoreType.DMA,
        ],
    )
    def agg_kernel(x_hbm, src_hbm, dst_hbm, out_hbm,
                   si0, si1, si2, si3, di0, di1, di2, di3, r0, r1, acc_sh,
                   g0, g1, is0, is1, is2, is3):
        cid = lax.axis_index("c")
        sid = lax.axis_index("s")
        wid = cid * NS + sid
        sidx = (si0, si1, si2, si3)
        didx = (di0, di1, di2, di3)
        rows = (r0, r1)
        gsem = (g0, g1)
        isem = (is0, is1, is2, is3)

        e0 = wid * CPT * CH

        def idx_start(i, q):
            pltpu.async_copy(
                src_hbm.at[pl.ds(e0 + i * CH, CH)], sidx[q], isem[q])
            pltpu.async_copy(
                dst_hbm.at[pl.ds(e0 + i * CH, CH)], didx[q], isem[q])

        def idx_wait(i, q):
            pltpu.make_async_copy(
                src_hbm.at[pl.ds(e0 + i * CH, CH)], sidx[q], isem[q]).wait()
            pltpu.make_async_copy(
                dst_hbm.at[pl.ds(e0 + i * CH, CH)], didx[q], isem[q]).wait()

        for q in range(4):
            idx_start(q, q)

        zeros16 = jnp.zeros((16,), jnp.float32)

        def zero_body(i, _):
            r0[i // (D // 16), pl.ds((i % (D // 16)) * 16, 16)] = zeros16
            return 0

        lax.fori_loop(0, ZR * (D // 16), zero_body, 0)

        for j in range(zchunks_per_tile):
            c = sid + j * NS

            @pl.when(c < n_row_chunks)
            def _():
                pltpu.sync_copy(r0.at[pl.ds(0, ZR)],
                                acc_sh.at[pl.ds(c * ZR, ZR)])

        idx_wait(0, 0)
        pltpu.async_copy(x_hbm.at[sidx[0]], r0, gsem[0])

        plsc.subcore_barrier()

        def outer(o, _):
            for p in range(4):
                i = 4 * o + p
                rp = p % 2
                pltpu.make_async_copy(
                    x_hbm.at[sidx[p]], rows[rp], gsem[rp]).wait()
                if p < 3:
                    idx_wait(i + 1, p + 1)
                    pltpu.async_copy(
                        x_hbm.at[sidx[p + 1]], rows[1 - rp], gsem[1 - rp])
                else:
                    @pl.when(o < CPT // 4 - 1)
                    def _():
                        idx_wait(i + 1, 0)
                        pltpu.async_copy(
                            x_hbm.at[sidx[0]], rows[1 - rp], gsem[1 - rp])
                pltpu.sync_copy(
                    rows[rp], acc_sh.at[didx[p]], add=True)

                @pl.when(o < CPT // 4 - 1)
                def _():
                    idx_start(i + 4, p)
            return 0

        lax.fori_loop(0, CPT // 4, outer, 0)
        plsc.subcore_barrier()

        for j in range(zchunks_per_tile):
            c = sid + j * NS

            @pl.when(c < n_row_chunks)
            def _():
                pltpu.sync_copy(acc_sh.at[pl.ds(c * ZR, ZR)],
                                r0.at[pl.ds(0, ZR)])
                pltpu.sync_copy(r0.at[pl.ds(0, ZR)],
                                out_hbm.at[cid, pl.ds(c * ZR, ZR)])

    return agg_kernel, NW * CPT * CH


def _mlp_call(x, p, W1, b1, W2, b2):
    N, D = x.shape
    BLK = 2000
    assert N % BLK == 0

    def mlp_body(x_ref, p0_ref, p1_ref, w1_ref, b1_ref, w2_ref, b2_ref,
                 o_ref):
        h = x_ref[...] + p0_ref[...] + p1_ref[...]
        h = jnp.dot(h, w1_ref[...], preferred_element_type=jnp.float32)
        h = jnp.maximum(h + b1_ref[...], 0.0)
        h = jnp.dot(h, w2_ref[...], preferred_element_type=jnp.float32)
        o_ref[...] = h + b2_ref[...]

    return pl.pallas_call(
        mlp_body,
        grid=(N // BLK,),
        in_specs=[
            pl.BlockSpec((BLK, D), lambda i: (i, 0)),
            pl.BlockSpec((BLK, D), lambda i: (i, 0)),
            pl.BlockSpec((BLK, D), lambda i: (i, 0)),
            pl.BlockSpec((D, D), lambda i: (0, 0)),
            pl.BlockSpec((1, D), lambda i: (0, 0)),
            pl.BlockSpec((D, D), lambda i: (0, 0)),
            pl.BlockSpec((1, D), lambda i: (0, 0)),
        ],
        out_specs=pl.BlockSpec((BLK, D), lambda i: (i, 0)),
        out_shape=jax.ShapeDtypeStruct((N, D), jnp.float32),
    )(x, p[0], p[1], W1, b1.reshape(1, D), W2, b2.reshape(1, D))


def kernel(x, edge_index, W1, b1, W2, b2):
    N, D = x.shape
    E = edge_index.shape[1]
    src = edge_index[0].astype(jnp.int32)
    dst = edge_index[1].astype(jnp.int32)
    agg_fn, E_pad = _make_agg(N, D, E)
    npad = E_pad - E
    src_p = jnp.concatenate([src, jnp.zeros((npad,), jnp.int32)])
    dst_p = jnp.concatenate([dst, jnp.full((npad,), N, jnp.int32)])
    p = agg_fn(x, src_p, dst_p)
    return _mlp_call(x, p, W1, b1, W2, b2)


# speedup vs baseline: 2.9170x; 2.9170x over previous
import functools

import jax
import jax.numpy as jnp
from jax import lax
from jax.experimental import pallas as pl
from jax.experimental.pallas import tpu as pltpu
from jax.experimental.pallas import tpu_sc as plsc


def _make_agg(N, D, E):
    info = plsc.get_sparse_core_info()
    NC, NS = info.num_cores, info.num_subcores
    NW = NC * NS
    CH = 128
    CPT = 80
    n_total = NW * CPT
    assert n_total * CH >= E
    TRASH = 248
    ZR = 80
    n_row_chunks = N // ZR
    assert n_row_chunks * ZR == N
    zchunks_per_tile = -(-n_row_chunks // NS)

    mesh = plsc.VectorSubcoreMesh(core_axis_name="c", subcore_axis_name="s")

    @functools.partial(
        pl.kernel,
        out_type=jax.ShapeDtypeStruct((NC, N, D), jnp.float32),
        mesh=mesh,
        scratch_types=[
            pltpu.VMEM((CH,), jnp.int32),
            pltpu.VMEM((CH,), jnp.int32),
            pltpu.VMEM((CH,), jnp.int32),
            pltpu.VMEM((CH,), jnp.int32),
            pltpu.VMEM((CH,), jnp.int32),
            pltpu.VMEM((CH,), jnp.int32),
            pltpu.VMEM((CH,), jnp.int32),
            pltpu.VMEM((CH,), jnp.int32),
            pltpu.VMEM((CH, D), jnp.float32),
            pltpu.VMEM((CH, D), jnp.float32),
            pltpu.VMEM_SHARED((N + TRASH, D), jnp.float32),
            pltpu.SemaphoreType.DMA,
            pltpu.SemaphoreType.DMA,
            pltpu.SemaphoreType.DMA,
            pltpu.SemaphoreType.DMA,
            pltpu.SemaphoreType.DMA,
            pltpu.SemaphoreType.DMA,
        ],
    )
    def agg_kernel(x_hbm, src_hbm, dst_hbm, out_hbm,
                   si0, si1, si2, si3, di0, di1, di2, di3, r0, r1, acc_sh,
                   g0, g1, is0, is1, is2, is3):
        cid = lax.axis_index("c")
        sid = lax.axis_index("s")
        wid = cid * NS + sid
        sidx = (si0, si1, si2, si3)
        didx = (di0, di1, di2, di3)
        rows = (r0, r1)
        gsem = (g0, g1)
        isem = (is0, is1, is2, is3)

        e0 = wid * CPT * CH

        def idx_start(i, q):
            pltpu.async_copy(
                src_hbm.at[pl.ds(e0 + i * CH, CH)], sidx[q], isem[q])
            pltpu.async_copy(
                dst_hbm.at[pl.ds(e0 + i * CH, CH)], didx[q], isem[q])

        def idx_wait(i, q):
            pltpu.make_async_copy(
                src_hbm.at[pl.ds(e0 + i * CH, CH)], sidx[q], isem[q]).wait()
            pltpu.make_async_copy(
                dst_hbm.at[pl.ds(e0 + i * CH, CH)], didx[q], isem[q]).wait()

        for q in range(4):
            idx_start(q, q)

        zeros16 = jnp.zeros((16,), jnp.float32)

        def zero_body(i, _):
            r0[i // (D // 16), pl.ds((i % (D // 16)) * 16, 16)] = zeros16
            return 0

        lax.fori_loop(0, ZR * (D // 16), zero_body, 0)

        for j in range(zchunks_per_tile):
            c = sid + j * NS

            @pl.when(c < n_row_chunks)
            def _():
                pltpu.sync_copy(r0.at[pl.ds(0, ZR)],
                                acc_sh.at[pl.ds(c * ZR, ZR)])

        idx_wait(0, 0)
        pltpu.async_copy(x_hbm.at[sidx[0]], r0, gsem[0])

        plsc.subcore_barrier()

        def outer(o, _):
            for p in range(4):
                i = 4 * o + p
                rp = p % 2
                pltpu.make_async_copy(
                    x_hbm.at[sidx[p]], rows[rp], gsem[rp]).wait()
                if p < 3:
                    idx_wait(i + 1, p + 1)
                    pltpu.async_copy(
                        x_hbm.at[sidx[p + 1]], rows[1 - rp], gsem[1 - rp])
                else:
                    @pl.when(o < CPT // 4 - 1)
                    def _():
                        idx_wait(i + 1, 0)
                        pltpu.async_copy(
                            x_hbm.at[sidx[0]], rows[1 - rp], gsem[1 - rp])
                pltpu.sync_copy(
                    rows[rp], acc_sh.at[didx[p]], add=True)

                @pl.when(o < CPT // 4 - 1)
                def _():
                    idx_start(i + 4, p)
            return 0

        lax.fori_loop(0, CPT // 4, outer, 0)
        plsc.subcore_barrier()

        for j in range(zchunks_per_tile):
            c = sid + j * NS

            @pl.when(c < n_row_chunks)
            def _():
                pltpu.sync_copy(acc_sh.at[pl.ds(c * ZR, ZR)],
                                r0.at[pl.ds(0, ZR)])
                pltpu.sync_copy(r0.at[pl.ds(0, ZR)],
                                out_hbm.at[cid, pl.ds(c * ZR, ZR)])

    return agg_kernel, NW, CPT * CH, TRASH


def _mlp_call(x, p, W1, b1, W2, b2):
    N, D = x.shape
    BLK = 2000
    assert N % BLK == 0

    def mlp_body(x_ref, p0_ref, p1_ref, w1_ref, b1_ref, w2_ref, b2_ref,
                 o_ref):
        h = x_ref[...] + p0_ref[...] + p1_ref[...]
        h = jnp.dot(h, w1_ref[...], preferred_element_type=jnp.float32)
        h = jnp.maximum(h + b1_ref[...], 0.0)
        h = jnp.dot(h, w2_ref[...], preferred_element_type=jnp.float32)
        o_ref[...] = h + b2_ref[...]

    return pl.pallas_call(
        mlp_body,
        grid=(N // BLK,),
        in_specs=[
            pl.BlockSpec((BLK, D), lambda i: (i, 0)),
            pl.BlockSpec((BLK, D), lambda i: (i, 0)),
            pl.BlockSpec((BLK, D), lambda i: (i, 0)),
            pl.BlockSpec((D, D), lambda i: (0, 0)),
            pl.BlockSpec((1, D), lambda i: (0, 0)),
            pl.BlockSpec((D, D), lambda i: (0, 0)),
            pl.BlockSpec((1, D), lambda i: (0, 0)),
        ],
        out_specs=pl.BlockSpec((BLK, D), lambda i: (i, 0)),
        out_shape=jax.ShapeDtypeStruct((N, D), jnp.float32),
    )(x, p[0], p[1], W1, b1.reshape(1, D), W2, b2.reshape(1, D))


def kernel(x, edge_index, W1, b1, W2, b2):
    N, D = x.shape
    E = edge_index.shape[1]
    src = edge_index[0].astype(jnp.int32)
    dst = edge_index[1].astype(jnp.int32)
    agg_fn, NW, ept, TRASH = _make_agg(N, D, E)
    per_tile = E // NW
    ppt = ept - per_tile
    lanes = jnp.arange(ppt, dtype=jnp.int32)
    pad_src = jnp.broadcast_to(lanes % N, (NW, ppt))
    pad_dst = jnp.broadcast_to(N + lanes % TRASH, (NW, ppt))
    src_p = jnp.concatenate(
        [src.reshape(NW, per_tile), pad_src], axis=1).reshape(-1)
    dst_p = jnp.concatenate(
        [dst.reshape(NW, per_tile), pad_dst], axis=1).reshape(-1)
    p = agg_fn(x, src_p, dst_p)
    return _mlp_call(x, p, W1, b1, W2, b2)


# speedup vs baseline: 2.9314x; 1.0049x over previous
import functools

import jax
import jax.numpy as jnp
from jax import lax
from jax.experimental import pallas as pl
from jax.experimental.pallas import tpu as pltpu
from jax.experimental.pallas import tpu_sc as plsc


def _make_agg(N, D, E):
    info = plsc.get_sparse_core_info()
    NC, NS = info.num_cores, info.num_subcores
    NW = NC * NS
    CH = 128
    GSZ = 8
    GROUPS = 10
    CPT = GSZ * GROUPS
    assert NW * CPT * CH >= E
    TRASH = 248
    ZR = 80
    n_row_chunks = N // ZR
    assert n_row_chunks * ZR == N
    zchunks_per_tile = -(-n_row_chunks // NS)

    mesh = plsc.VectorSubcoreMesh(core_axis_name="c", subcore_axis_name="s")

    @functools.partial(
        pl.kernel,
        out_type=jax.ShapeDtypeStruct((NC, N, D), jnp.float32),
        mesh=mesh,
        scratch_types=[
            pltpu.VMEM((GSZ, CH), jnp.int32),
            pltpu.VMEM((GSZ, CH), jnp.int32),
            pltpu.VMEM((GSZ, CH), jnp.int32),
            pltpu.VMEM((GSZ, CH), jnp.int32),
            pltpu.VMEM((CH, D), jnp.float32),
            pltpu.VMEM((CH, D), jnp.float32),
            pltpu.VMEM_SHARED((N + TRASH, D), jnp.float32),
            pltpu.SemaphoreType.DMA,
            pltpu.SemaphoreType.DMA,
            pltpu.SemaphoreType.DMA,
            pltpu.SemaphoreType.DMA,
        ],
    )
    def agg_kernel(x_hbm, src_hbm, dst_hbm, out_hbm,
                   sb0, sb1, db0, db1, r0, r1, acc_sh,
                   g0, g1, is0, is1):
        cid = lax.axis_index("c")
        sid = lax.axis_index("s")
        wid = cid * NS + sid
        sbuf = (sb0, sb1)
        dbuf = (db0, db1)
        rows = (r0, r1)
        gsem = (g0, g1)
        isem = (is0, is1)

        c0 = wid * CPT

        def idx_start(g, gb):
            pltpu.async_copy(
                src_hbm.at[pl.ds(c0 + g * GSZ, GSZ)], sbuf[gb], isem[gb])
            pltpu.async_copy(
                dst_hbm.at[pl.ds(c0 + g * GSZ, GSZ)], dbuf[gb], isem[gb])

        def idx_wait(g, gb):
            pltpu.make_async_copy(
                src_hbm.at[pl.ds(c0 + g * GSZ, GSZ)], sbuf[gb],
                isem[gb]).wait()
            pltpu.make_async_copy(
                dst_hbm.at[pl.ds(c0 + g * GSZ, GSZ)], dbuf[gb],
                isem[gb]).wait()

        idx_start(0, 0)
        idx_start(1, 1)

        zeros16 = jnp.zeros((16,), jnp.float32)

        def zero_body(i, _):
            r0[i // (D // 16), pl.ds((i % (D // 16)) * 16, 16)] = zeros16
            return 0

        lax.fori_loop(0, ZR * (D // 16), zero_body, 0)

        for j in range(zchunks_per_tile):
            c = sid + j * NS

            @pl.when(c < n_row_chunks)
            def _():
                pltpu.sync_copy(r0.at[pl.ds(0, ZR)],
                                acc_sh.at[pl.ds(c * ZR, ZR)])

        idx_wait(0, 0)
        pltpu.async_copy(x_hbm.at[sb0.at[0]], r0, gsem[0])

        plsc.subcore_barrier()

        def outer(og, _):
            for gb in range(2):
                g = 2 * og + gb
                for j in range(GSZ):
                    rp = j % 2
                    pltpu.make_async_copy(
                        x_hbm.at[sbuf[gb].at[j]], rows[rp], gsem[rp]).wait()
                    if j < GSZ - 1:
                        pltpu.async_copy(
                            x_hbm.at[sbuf[gb].at[j + 1]], rows[1 - rp],
                            gsem[1 - rp])
                    else:
                        cond = (og < (GROUPS // 2) - 1) if gb == 1 else True

                        def next_group():
                            idx_wait(g + 1, 1 - gb)
                            pltpu.async_copy(
                                x_hbm.at[sbuf[1 - gb].at[0]], rows[1 - rp],
                                gsem[1 - rp])

                        if cond is True:
                            next_group()
                        else:
                            pl.when(og < (GROUPS // 2) - 1)(next_group)
                    pltpu.sync_copy(
                        rows[rp], acc_sh.at[dbuf[gb].at[j]], add=True)

                @pl.when(og < (GROUPS // 2) - 1)
                def _():
                    idx_start(g + 2, gb)
            return 0

        lax.fori_loop(0, GROUPS // 2, outer, 0)
        plsc.subcore_barrier()

        for j in range(zchunks_per_tile):
            c = sid + j * NS

            @pl.when(c < n_row_chunks)
            def _():
                pltpu.sync_copy(acc_sh.at[pl.ds(c * ZR, ZR)],
                                out_hbm.at[cid, pl.ds(c * ZR, ZR)])

    return agg_kernel, NW, CPT * CH, TRASH, CH


def _mlp_call(x, p, W1, b1, W2, b2):
    N, D = x.shape
    BLK = 2000
    assert N % BLK == 0

    def mlp_body(x_ref, p0_ref, p1_ref, w1_ref, b1_ref, w2_ref, b2_ref,
                 o_ref):
        h = x_ref[...] + p0_ref[...] + p1_ref[...]
        h = jnp.dot(h, w1_ref[...], preferred_element_type=jnp.float32)
        h = jnp.maximum(h + b1_ref[...], 0.0)
        h = jnp.dot(h, w2_ref[...], preferred_element_type=jnp.float32)
        o_ref[...] = h + b2_ref[...]

    return pl.pallas_call(
        mlp_body,
        grid=(N // BLK,),
        in_specs=[
            pl.BlockSpec((BLK, D), lambda i: (i, 0)),
            pl.BlockSpec((BLK, D), lambda i: (i, 0)),
            pl.BlockSpec((BLK, D), lambda i: (i, 0)),
            pl.BlockSpec((D, D), lambda i: (0, 0)),
            pl.BlockSpec((1, D), lambda i: (0, 0)),
            pl.BlockSpec((D, D), lambda i: (0, 0)),
            pl.BlockSpec((1, D), lambda i: (0, 0)),
        ],
        out_specs=pl.BlockSpec((BLK, D), lambda i: (i, 0)),
        out_shape=jax.ShapeDtypeStruct((N, D), jnp.float32),
    )(x, p[0], p[1], W1, b1.reshape(1, D), W2, b2.reshape(1, D))


def kernel(x, edge_index, W1, b1, W2, b2):
    N, D = x.shape
    E = edge_index.shape[1]
    src = edge_index[0].astype(jnp.int32)
    dst = edge_index[1].astype(jnp.int32)
    agg_fn, NW, ept, TRASH, CH = _make_agg(N, D, E)
    per_tile = E // NW
    ppt = ept - per_tile
    lanes = jnp.arange(ppt, dtype=jnp.int32)
    pad_src = jnp.broadcast_to(lanes % N, (NW, ppt))
    pad_dst = jnp.broadcast_to(N + lanes % TRASH, (NW, ppt))
    src_p = jnp.concatenate(
        [src.reshape(NW, per_tile), pad_src], axis=1).reshape(-1, CH)
    dst_p = jnp.concatenate(
        [dst.reshape(NW, per_tile), pad_dst], axis=1).reshape(-1, CH)
    p = agg_fn(x, src_p, dst_p)
    return _mlp_call(x, p, W1, b1, W2, b2)


# speedup vs baseline: 3.5918x; 1.2253x over previous
import functools

import jax
import jax.numpy as jnp
from jax import lax
from jax.experimental import pallas as pl
from jax.experimental.pallas import tpu as pltpu
from jax.experimental.pallas import tpu_sc as plsc


def _make_agg(N, D, E):
    info = plsc.get_sparse_core_info()
    NC, NS = info.num_cores, info.num_subcores
    NW = NC * NS
    CH = 88
    CPT = 116
    assert NW * CPT * CH >= E
    NBUF = 4
    TRASH = 248
    ZR = 80
    n_row_chunks = N // ZR
    assert n_row_chunks * ZR == N
    zchunks_per_tile = -(-n_row_chunks // NS)

    mesh = plsc.VectorSubcoreMesh(core_axis_name="c", subcore_axis_name="s")

    @functools.partial(
        pl.kernel,
        out_type=jax.ShapeDtypeStruct((NC, N, D), jnp.float32),
        mesh=mesh,
        scratch_types=[
            pltpu.VMEM((CH,), jnp.int32),
            pltpu.VMEM((CH,), jnp.int32),
            pltpu.VMEM((CH,), jnp.int32),
            pltpu.VMEM((CH,), jnp.int32),
            pltpu.VMEM((CH,), jnp.int32),
            pltpu.VMEM((CH,), jnp.int32),
            pltpu.VMEM((CH,), jnp.int32),
            pltpu.VMEM((CH,), jnp.int32),
            pltpu.VMEM((CH, D), jnp.float32),
            pltpu.VMEM((CH, D), jnp.float32),
            pltpu.VMEM((CH, D), jnp.float32),
            pltpu.VMEM((CH, D), jnp.float32),
            pltpu.VMEM_SHARED((N + TRASH, D), jnp.float32),
            pltpu.SemaphoreType.DMA,
            pltpu.SemaphoreType.DMA,
            pltpu.SemaphoreType.DMA,
            pltpu.SemaphoreType.DMA,
            pltpu.SemaphoreType.DMA,
            pltpu.SemaphoreType.DMA,
            pltpu.SemaphoreType.DMA,
            pltpu.SemaphoreType.DMA,
        ],
    )
    def agg_kernel(x_hbm, src_hbm, dst_hbm, out_hbm,
                   si0, si1, si2, si3, di0, di1, di2, di3,
                   r0, r1, r2, r3, acc_sh,
                   g0, g1, g2, g3, is0, is1, is2, is3):
        cid = lax.axis_index("c")
        sid = lax.axis_index("s")
        wid = cid * NS + sid
        sidx = (si0, si1, si2, si3)
        didx = (di0, di1, di2, di3)
        rows = (r0, r1, r2, r3)
        gsem = (g0, g1, g2, g3)
        isem = (is0, is1, is2, is3)

        e0 = wid * CPT * CH

        def idx_start(i, q):
            pltpu.async_copy(
                src_hbm.at[pl.ds(e0 + i * CH, CH)], sidx[q], isem[q])
            pltpu.async_copy(
                dst_hbm.at[pl.ds(e0 + i * CH, CH)], didx[q], isem[q])

        def idx_wait(i, q):
            pltpu.make_async_copy(
                src_hbm.at[pl.ds(e0 + i * CH, CH)], sidx[q], isem[q]).wait()
            pltpu.make_async_copy(
                dst_hbm.at[pl.ds(e0 + i * CH, CH)], didx[q], isem[q]).wait()

        for q in range(NBUF):
            idx_start(q, q)

        zeros16 = jnp.zeros((16,), jnp.float32)

        def zero_body(i, _):
            r0[i // (D // 16), pl.ds((i % (D // 16)) * 16, 16)] = zeros16
            return 0

        lax.fori_loop(0, ZR * (D // 16), zero_body, 0)

        for j in range(zchunks_per_tile):
            c = sid + j * NS

            @pl.when(c < n_row_chunks)
            def _():
                pltpu.sync_copy(r0.at[pl.ds(0, ZR)],
                                acc_sh.at[pl.ds(c * ZR, ZR)])

        for q in range(NBUF - 1):
            idx_wait(q, q)
            pltpu.async_copy(x_hbm.at[sidx[q]], rows[q], gsem[q])

        plsc.subcore_barrier()

        n_outer = CPT // NBUF

        def outer(o, _):
            for p in range(NBUF):
                i = NBUF * o + p
                b3 = (p + 3) % NBUF
                pltpu.make_async_copy(
                    x_hbm.at[sidx[p]], rows[p], gsem[p]).wait()
                pltpu.sync_copy(
                    rows[p], acc_sh.at[didx[p]], add=True)

                @pl.when(o < n_outer - 1)
                def _():
                    idx_start(i + NBUF, p)

                def next_gather():
                    idx_wait(i + 3, b3)
                    pltpu.async_copy(
                        x_hbm.at[sidx[b3]], rows[b3], gsem[b3])

                if p == 0:
                    next_gather()
                else:
                    pl.when(o < n_outer - 1)(next_gather)
            return 0

        lax.fori_loop(0, n_outer, outer, 0)
        plsc.subcore_barrier()

        for j in range(zchunks_per_tile):
            c = sid + j * NS

            @pl.when(c < n_row_chunks)
            def _():
                pltpu.sync_copy(acc_sh.at[pl.ds(c * ZR, ZR)],
                                out_hbm.at[cid, pl.ds(c * ZR, ZR)])

    return agg_kernel, NW, CPT * CH, TRASH


def _mlp_call(x, p, W1, b1, W2, b2):
    N, D = x.shape
    BLK = 2000
    assert N % BLK == 0

    def mlp_body(x_ref, p0_ref, p1_ref, w1_ref, b1_ref, w2_ref, b2_ref,
                 o_ref):
        h = x_ref[...] + p0_ref[...] + p1_ref[...]
        h = jnp.dot(h, w1_ref[...], preferred_element_type=jnp.float32)
        h = jnp.maximum(h + b1_ref[...], 0.0)
        h = jnp.dot(h, w2_ref[...], preferred_element_type=jnp.float32)
        o_ref[...] = h + b2_ref[...]

    return pl.pallas_call(
        mlp_body,
        grid=(N // BLK,),
        in_specs=[
            pl.BlockSpec((BLK, D), lambda i: (i, 0)),
            pl.BlockSpec((BLK, D), lambda i: (i, 0)),
            pl.BlockSpec((BLK, D), lambda i: (i, 0)),
            pl.BlockSpec((D, D), lambda i: (0, 0)),
            pl.BlockSpec((1, D), lambda i: (0, 0)),
            pl.BlockSpec((D, D), lambda i: (0, 0)),
            pl.BlockSpec((1, D), lambda i: (0, 0)),
        ],
        out_specs=pl.BlockSpec((BLK, D), lambda i: (i, 0)),
        out_shape=jax.ShapeDtypeStruct((N, D), jnp.float32),
    )(x, p[0], p[1], W1, b1.reshape(1, D), W2, b2.reshape(1, D))


def kernel(x, edge_index, W1, b1, W2, b2):
    N, D = x.shape
    E = edge_index.shape[1]
    src = edge_index[0].astype(jnp.int32)
    dst = edge_index[1].astype(jnp.int32)
    agg_fn, NW, ept, TRASH = _make_agg(N, D, E)
    per_tile = E // NW
    ppt = ept - per_tile
    lanes = jnp.arange(ppt, dtype=jnp.int32)[None, :]
    tiles = jnp.arange(NW, dtype=jnp.int32)[:, None]
    pad_src = jnp.broadcast_to(lanes % N, (NW, ppt))
    pad_dst = N + (lanes + tiles * 8) % TRASH
    src_p = jnp.concatenate(
        [src.reshape(NW, per_tile), pad_src], axis=1).reshape(-1)
    dst_p = jnp.concatenate(
        [dst.reshape(NW, per_tile), pad_dst], axis=1).reshape(-1)
    p = agg_fn(x, src_p, dst_p)
    return _mlp_call(x, p, W1, b1, W2, b2)


# speedup vs baseline: 3.6630x; 1.0198x over previous
import functools

import jax
import jax.numpy as jnp
from jax import lax
from jax.experimental import pallas as pl
from jax.experimental.pallas import tpu as pltpu
from jax.experimental.pallas import tpu_sc as plsc


def _make_agg(N, D, E):
    info = plsc.get_sparse_core_info()
    NC, NS, L = info.num_cores, info.num_subcores, info.num_lanes
    NW = NC * NS
    CH = 80
    REAL = E // (NW * CH)
    assert REAL * NW * CH == E
    NBUF = 4
    NPAD = 3
    CPT = REAL + NPAD
    assert CPT % NBUF == 0
    TRASH = 248
    ZR = 80
    n_row_chunks = N // ZR
    assert n_row_chunks * ZR == N
    zchunks_per_tile = -(-n_row_chunks // NS)

    mesh = plsc.VectorSubcoreMesh(core_axis_name="c", subcore_axis_name="s")

    @functools.partial(
        pl.kernel,
        out_type=jax.ShapeDtypeStruct((NC, N, D), jnp.float32),
        mesh=mesh,
        scratch_types=[
            pltpu.VMEM((CH,), jnp.int32),
            pltpu.VMEM((CH,), jnp.int32),
            pltpu.VMEM((CH,), jnp.int32),
            pltpu.VMEM((CH,), jnp.int32),
            pltpu.VMEM((CH,), jnp.int32),
            pltpu.VMEM((CH,), jnp.int32),
            pltpu.VMEM((CH,), jnp.int32),
            pltpu.VMEM((CH,), jnp.int32),
            pltpu.VMEM((CH,), jnp.int32),
            pltpu.VMEM((CH,), jnp.int32),
            pltpu.VMEM((CH,), jnp.int32),
            pltpu.VMEM((CH,), jnp.int32),
            pltpu.VMEM((CH,), jnp.int32),
            pltpu.VMEM((CH,), jnp.int32),
            pltpu.VMEM((CH, D), jnp.float32),
            pltpu.VMEM((CH, D), jnp.float32),
            pltpu.VMEM((CH, D), jnp.float32),
            pltpu.VMEM((CH, D), jnp.float32),
            pltpu.VMEM_SHARED((N + TRASH, D), jnp.float32),
            pltpu.SemaphoreType.DMA,
            pltpu.SemaphoreType.DMA,
            pltpu.SemaphoreType.DMA,
            pltpu.SemaphoreType.DMA,
            pltpu.SemaphoreType.DMA,
            pltpu.SemaphoreType.DMA,
            pltpu.SemaphoreType.DMA,
            pltpu.SemaphoreType.DMA,
        ],
    )
    def agg_kernel(x_hbm, src_hbm, dst_hbm, out_hbm,
                   si0, si1, si2, si3, di0, di1, di2, di3,
                   ps0, ps1, ps2, pd0, pd1, pd2,
                   r0, r1, r2, r3, acc_sh,
                   g0, g1, g2, g3, is0, is1, is2, is3):
        cid = lax.axis_index("c")
        sid = lax.axis_index("s")
        wid = cid * NS + sid
        sidx = (si0, si1, si2, si3)
        didx = (di0, di1, di2, di3)
        psrc = (ps0, ps1, ps2)
        pdst = (pd0, pd1, pd2)
        rows = (r0, r1, r2, r3)
        gsem = (g0, g1, g2, g3)
        isem = (is0, is1, is2, is3)

        e0 = wid * REAL * CH

        def idx_start(i, q):
            pltpu.async_copy(
                src_hbm.at[pl.ds(e0 + i * CH, CH)], sidx[q], isem[q])
            pltpu.async_copy(
                dst_hbm.at[pl.ds(e0 + i * CH, CH)], didx[q], isem[q])

        def idx_wait(i, q):
            pltpu.make_async_copy(
                src_hbm.at[pl.ds(e0 + i * CH, CH)], sidx[q], isem[q]).wait()
            pltpu.make_async_copy(
                dst_hbm.at[pl.ds(e0 + i * CH, CH)], didx[q], isem[q]).wait()

        for q in range(NBUF):
            idx_start(q, q)

        lane = lax.iota(jnp.int32, L)
        for k in range(NPAD):
            for g in range(CH // L):
                off = lane + (k * CH + g * L)
                psrc[k][pl.ds(g * L, L)] = (off * 37 + wid) % N
                pdst[k][pl.ds(g * L, L)] = N + (off + wid * 8) % TRASH

        zeros16 = jnp.zeros((L,), jnp.float32)

        def zero_body(i, _):
            r0[i // (D // L), pl.ds((i % (D // L)) * L, L)] = zeros16
            return 0

        lax.fori_loop(0, ZR * (D // L), zero_body, 0)

        for j in range(zchunks_per_tile):
            c = sid + j * NS

            @pl.when(c < n_row_chunks)
            def _():
                pltpu.sync_copy(r0, acc_sh.at[pl.ds(c * ZR, ZR)])

        for q in range(NBUF - 1):
            idx_wait(q, q)
            pltpu.async_copy(x_hbm.at[sidx[q]], rows[q], gsem[q])

        plsc.subcore_barrier()

        n_outer = (CPT - NBUF) // NBUF

        def outer(o, _):
            for p in range(NBUF):
                i = NBUF * o + p
                b3 = (p + 3) % NBUF
                pltpu.make_async_copy(
                    x_hbm.at[sidx[p]], rows[p], gsem[p]).wait()
                pltpu.sync_copy(
                    rows[p], acc_sh.at[didx[p]], add=True)

                if p == 0:
                    idx_start(i + NBUF, p)
                else:
                    @pl.when(o < n_outer - 1)
                    def _():
                        idx_start(i + NBUF, p)

                def next_gather():
                    idx_wait(i + 3, b3)
                    pltpu.async_copy(
                        x_hbm.at[sidx[b3]], rows[b3], gsem[b3])

                if p <= 1:
                    next_gather()
                else:
                    pl.when(o < n_outer - 1)(next_gather)

                    @pl.when(o == n_outer - 1)
                    def _():
                        pltpu.async_copy(
                            x_hbm.at[psrc[p - 2]], rows[b3], gsem[b3])
            return 0

        lax.fori_loop(0, n_outer, outer, 0)

        pltpu.async_copy(x_hbm.at[psrc[2]], rows[3], gsem[3])
        pltpu.make_async_copy(x_hbm.at[sidx[0]], rows[0], gsem[0]).wait()
        pltpu.sync_copy(rows[0], acc_sh.at[didx[0]], add=True)
        for k in range(NPAD):
            b = k + 1
            pltpu.make_async_copy(
                x_hbm.at[psrc[k]], rows[b], gsem[b]).wait()
            pltpu.sync_copy(rows[b], acc_sh.at[pdst[k]], add=True)

        plsc.subcore_barrier()

        for j in range(zchunks_per_tile):
            c = sid + j * NS

            @pl.when(c < n_row_chunks)
            def _():
                pltpu.sync_copy(acc_sh.at[pl.ds(c * ZR, ZR)],
                                out_hbm.at[cid, pl.ds(c * ZR, ZR)])

    return agg_kernel


def _mlp_call(x, p, W1, b1, W2, b2):
    N, D = x.shape
    BLK = 2000
    assert N % BLK == 0

    def mlp_body(x_ref, p0_ref, p1_ref, w1_ref, b1_ref, w2_ref, b2_ref,
                 o_ref):
        h = x_ref[...] + p0_ref[...] + p1_ref[...]
        h = jnp.dot(h, w1_ref[...], preferred_element_type=jnp.float32)
        h = jnp.maximum(h + b1_ref[...], 0.0)
        h = jnp.dot(h, w2_ref[...], preferred_element_type=jnp.float32)
        o_ref[...] = h + b2_ref[...]

    return pl.pallas_call(
        mlp_body,
        grid=(N // BLK,),
        in_specs=[
            pl.BlockSpec((BLK, D), lambda i: (i, 0)),
            pl.BlockSpec((BLK, D), lambda i: (i, 0)),
            pl.BlockSpec((BLK, D), lambda i: (i, 0)),
            pl.BlockSpec((D, D), lambda i: (0, 0)),
            pl.BlockSpec((1, D), lambda i: (0, 0)),
            pl.BlockSpec((D, D), lambda i: (0, 0)),
            pl.BlockSpec((1, D), lambda i: (0, 0)),
        ],
        out_specs=pl.BlockSpec((BLK, D), lambda i: (i, 0)),
        out_shape=jax.ShapeDtypeStruct((N, D), jnp.float32),
    )(x, p[0], p[1], W1, b1.reshape(1, D), W2, b2.reshape(1, D))


def kernel(x, edge_index, W1, b1, W2, b2):
    N, D = x.shape
    E = edge_index.shape[1]
    src = edge_index[0].astype(jnp.int32)
    dst = edge_index[1].astype(jnp.int32)
    p = _make_agg(N, D, E)(x, src, dst)
    return _mlp_call(x, p, W1, b1, W2, b2)


# speedup vs baseline: 4.1624x; 1.1363x over previous
import functools

import jax
import jax.numpy as jnp
from jax import lax
from jax.experimental import pallas as pl
from jax.experimental.pallas import tpu as pltpu
from jax.experimental.pallas import tpu_sc as plsc


def _make_agg(N, D, E):
    info = plsc.get_sparse_core_info()
    NC, NS, L = info.num_cores, info.num_subcores, info.num_lanes
    NW = NC * NS
    CH = 80
    REAL = E // (NW * CH)
    assert REAL * NW * CH == E
    NBUF = 4
    NPAD = 3
    CPT = REAL + NPAD
    assert CPT % NBUF == 0
    TRASH = 248
    ZR = 80
    n_row_chunks = N // ZR
    assert n_row_chunks * ZR == N
    zchunks_per_tile = -(-n_row_chunks // NS)

    mesh = plsc.VectorSubcoreMesh(core_axis_name="c", subcore_axis_name="s")

    @functools.partial(
        pl.kernel,
        out_type=jax.ShapeDtypeStruct((NC, N, D), jnp.float32),
        mesh=mesh,
        scratch_types=[
            pltpu.VMEM((CH,), jnp.int32),
            pltpu.VMEM((CH,), jnp.int32),
            pltpu.VMEM((CH,), jnp.int32),
            pltpu.VMEM((CH,), jnp.int32),
            pltpu.VMEM((CH,), jnp.int32),
            pltpu.VMEM((CH,), jnp.int32),
            pltpu.VMEM((CH,), jnp.int32),
            pltpu.VMEM((CH,), jnp.int32),
            pltpu.VMEM((CH,), jnp.int32),
            pltpu.VMEM((CH,), jnp.int32),
            pltpu.VMEM((CH,), jnp.int32),
            pltpu.VMEM((CH,), jnp.int32),
            pltpu.VMEM((CH,), jnp.int32),
            pltpu.VMEM((CH,), jnp.int32),
            pltpu.VMEM((CH, D), jnp.float32),
            pltpu.VMEM((CH, D), jnp.float32),
            pltpu.VMEM((CH, D), jnp.float32),
            pltpu.VMEM((CH, D), jnp.float32),
            pltpu.VMEM_SHARED((N + TRASH, D), jnp.float32),
            pltpu.SemaphoreType.DMA,
            pltpu.SemaphoreType.DMA,
            pltpu.SemaphoreType.DMA,
            pltpu.SemaphoreType.DMA,
            pltpu.SemaphoreType.DMA,
            pltpu.SemaphoreType.DMA,
            pltpu.SemaphoreType.DMA,
            pltpu.SemaphoreType.DMA,
        ],
    )
    def agg_kernel(x_hbm, edge_hbm, out_hbm,
                   si0, si1, si2, si3, di0, di1, di2, di3,
                   ps0, ps1, ps2, pd0, pd1, pd2,
                   r0, r1, r2, r3, acc_sh,
                   g0, g1, g2, g3, is0, is1, is2, is3):
        cid = lax.axis_index("c")
        sid = lax.axis_index("s")
        wid = cid * NS + sid
        sidx = (si0, si1, si2, si3)
        didx = (di0, di1, di2, di3)
        psrc = (ps0, ps1, ps2)
        pdst = (pd0, pd1, pd2)
        rows = (r0, r1, r2, r3)
        gsem = (g0, g1, g2, g3)
        isem = (is0, is1, is2, is3)

        e0 = wid * REAL * CH

        def idx_start(i, q):
            pltpu.async_copy(
                edge_hbm.at[pl.ds(e0 + i * CH, CH)], sidx[q], isem[q])
            pltpu.async_copy(
                edge_hbm.at[pl.ds(E + e0 + i * CH, CH)], didx[q], isem[q])

        def idx_wait(i, q):
            pltpu.make_async_copy(
                edge_hbm.at[pl.ds(e0 + i * CH, CH)], sidx[q],
                isem[q]).wait()
            pltpu.make_async_copy(
                edge_hbm.at[pl.ds(E + e0 + i * CH, CH)], didx[q],
                isem[q]).wait()

        for q in range(NBUF):
            idx_start(q, q)

        lane = lax.iota(jnp.int32, L)
        for k in range(NPAD):
            for g in range(CH // L):
                off = lane + (k * CH + g * L)
                psrc[k][pl.ds(g * L, L)] = (off * 37 + wid) % N
                pdst[k][pl.ds(g * L, L)] = N + (off + wid * 8) % TRASH

        zeros16 = jnp.zeros((L,), jnp.float32)

        def zero_body(i, _):
            r0[i // (D // L), pl.ds((i % (D // L)) * L, L)] = zeros16
            return 0

        lax.fori_loop(0, ZR * (D // L), zero_body, 0)

        for j in range(zchunks_per_tile):
            c = sid + j * NS

            @pl.when(c < n_row_chunks)
            def _():
                pltpu.sync_copy(r0, acc_sh.at[pl.ds(c * ZR, ZR)])

        for q in range(NBUF - 1):
            idx_wait(q, q)
            pltpu.async_copy(x_hbm.at[sidx[q]], rows[q], gsem[q])

        plsc.subcore_barrier()

        n_outer = (CPT - NBUF) // NBUF

        def outer(o, _):
            for p in range(NBUF):
                i = NBUF * o + p
                b3 = (p + 3) % NBUF
                pltpu.make_async_copy(
                    x_hbm.at[sidx[p]], rows[p], gsem[p]).wait()
                pltpu.sync_copy(
                    rows[p], acc_sh.at[didx[p]], add=True)

                if p == 0:
                    idx_start(i + NBUF, p)
                else:
                    @pl.when(o < n_outer - 1)
                    def _():
                        idx_start(i + NBUF, p)

                def next_gather():
                    idx_wait(i + 3, b3)
                    pltpu.async_copy(
                        x_hbm.at[sidx[b3]], rows[b3], gsem[b3])

                if p <= 1:
                    next_gather()
                else:
                    pl.when(o < n_outer - 1)(next_gather)

                    @pl.when(o == n_outer - 1)
                    def _():
                        pltpu.async_copy(
                            x_hbm.at[psrc[p - 2]], rows[b3], gsem[b3])
            return 0

        lax.fori_loop(0, n_outer, outer, 0)

        pltpu.async_copy(x_hbm.at[psrc[2]], rows[3], gsem[3])
        pltpu.make_async_copy(x_hbm.at[sidx[0]], rows[0], gsem[0]).wait()
        pltpu.sync_copy(rows[0], acc_sh.at[didx[0]], add=True)
        for k in range(NPAD):
            b = k + 1
            pltpu.make_async_copy(
                x_hbm.at[psrc[k]], rows[b], gsem[b]).wait()
            pltpu.sync_copy(rows[b], acc_sh.at[pdst[k]], add=True)

        plsc.subcore_barrier()

        for j in range(zchunks_per_tile):
            c = sid + j * NS

            @pl.when(c < n_row_chunks)
            def _():
                pltpu.sync_copy(acc_sh.at[pl.ds(c * ZR, ZR)],
                                out_hbm.at[cid, pl.ds(c * ZR, ZR)])

    return agg_kernel


def _mlp_call(x, p, W1, b1, W2, b2):
    N, D = x.shape
    BLK = 2000
    assert N % BLK == 0

    def mlp_body(x_ref, p_ref, w1_ref, b1_ref, w2_ref, b2_ref,
                 o_ref):
        h = x_ref[...] + p_ref[0] + p_ref[1]
        h = jnp.dot(h, w1_ref[...], preferred_element_type=jnp.float32)
        h = jnp.maximum(h + b1_ref[...], 0.0)
        h = jnp.dot(h, w2_ref[...], preferred_element_type=jnp.float32)
        o_ref[...] = h + b2_ref[...]

    return pl.pallas_call(
        mlp_body,
        grid=(N // BLK,),
        in_specs=[
            pl.BlockSpec((BLK, D), lambda i: (i, 0)),
            pl.BlockSpec((2, BLK, D), lambda i: (0, i, 0)),
            pl.BlockSpec((D, D), lambda i: (0, 0)),
            pl.BlockSpec((1, D), lambda i: (0, 0)),
            pl.BlockSpec((D, D), lambda i: (0, 0)),
            pl.BlockSpec((1, D), lambda i: (0, 0)),
        ],
        out_specs=pl.BlockSpec((BLK, D), lambda i: (i, 0)),
        out_shape=jax.ShapeDtypeStruct((N, D), jnp.float32),
    )(x, p, W1, b1.reshape(1, D), W2, b2.reshape(1, D))


def kernel(x, edge_index, W1, b1, W2, b2):
    N, D = x.shape
    E = edge_index.shape[1]
    edge_flat = edge_index.astype(jnp.int32).reshape(-1)
    p = _make_agg(N, D, E)(x, edge_flat)
    return _mlp_call(x, p, W1, b1, W2, b2)
